# baseline (speedup 1.0000x reference)
import functools

import jax
import jax.numpy as jnp
from jax import lax
from jax.experimental import pallas as pl
from jax.experimental.pallas import tpu as pltpu

_BN_EPS = 1e-5


def _conv_kernel(x_ref, w_ref, yconv_ref, stats_ref, *, ho, wo, kh, kw, c):
    xt = jnp.transpose(x_ref[0], (1, 2, 0)).astype(jnp.bfloat16)
    xp = jnp.pad(xt, ((1, 1), (1, 1), (0, 0)))
    cat = jnp.concatenate([xp[:, j:j + wo, :] for j in range(kw)], axis=-1)
    m = ho * wo
    acc = None
    for i in range(kh):
        lhs = cat[i:i + ho].reshape(m, kw * c)
        part = jnp.dot(lhs, w_ref[i], preferred_element_type=jnp.float32)
        acc = part if acc is None else acc + part
    stats_ref[0] = jnp.concatenate(
        [jnp.sum(acc, axis=0, keepdims=True),
         jnp.sum(acc * acc, axis=0, keepdims=True)], axis=0)
    yconv_ref[0] = acc.astype(jnp.bfloat16)


def _bn_kernel(yconv_ref, stats_ref, g_ref, b_ref, out_ref, *, m_total):
    mean = jnp.sum(stats_ref[:, 0, :], axis=0, keepdims=True) / m_total
    ex2 = jnp.sum(stats_ref[:, 1, :], axis=0, keepdims=True) / m_total
    var = jnp.maximum(ex2 - mean * mean, 0.0)
    scale = g_ref[...] * lax.rsqrt(var + _BN_EPS)
    shift = b_ref[...] - mean * scale
    y = jnp.maximum(yconv_ref[0].astype(jnp.float32) * scale + shift, 0.0)
    out_ref[0] = jnp.transpose(y, (1, 0))


@jax.jit
def _conv_bn_relu(x, weight, gamma, beta):
    n, c, h, w = x.shape
    co, _, kh, kw = weight.shape
    ho, wo = h, w
    m = ho * wo
    m_total = n * m

    w_cat = jnp.transpose(weight, (2, 3, 1, 0)).reshape(kh, kw * c, co)
    w_cat = w_cat.astype(jnp.bfloat16)
    g2 = gamma.reshape(1, co)
    b2 = beta.reshape(1, co)

    cparams = pltpu.CompilerParams(dimension_semantics=("parallel",))

    yconv, stats = pl.pallas_call(
        functools.partial(_conv_kernel, ho=ho, wo=wo, kh=kh, kw=kw, c=c),
        out_shape=(jax.ShapeDtypeStruct((n, m, co), jnp.bfloat16),
                   jax.ShapeDtypeStruct((n, 2, co), jnp.float32)),
        grid=(n,),
        in_specs=[pl.BlockSpec((1, c, h, w), lambda nb: (nb, 0, 0, 0)),
                  pl.BlockSpec((kh, kw * c, co), lambda nb: (0, 0, 0))],
        out_specs=(pl.BlockSpec((1, m, co), lambda nb: (nb, 0, 0)),
                   pl.BlockSpec((1, 2, co), lambda nb: (nb, 0, 0))),
        compiler_params=cparams,
    )(x, w_cat)

    out_cm = pl.pallas_call(
        functools.partial(_bn_kernel, m_total=m_total),
        out_shape=jax.ShapeDtypeStruct((n, co, m), jnp.float32),
        grid=(n,),
        in_specs=[pl.BlockSpec((1, m, co), lambda nb: (nb, 0, 0)),
                  pl.BlockSpec((n, 2, co), lambda nb: (0, 0, 0)),
                  pl.BlockSpec((1, co), lambda nb: (0, 0)),
                  pl.BlockSpec((1, co), lambda nb: (0, 0))],
        out_specs=pl.BlockSpec((1, co, m), lambda nb: (nb, 0, 0)),
        compiler_params=cparams,
    )(yconv, stats, g2, b2)

    return out_cm.reshape(n, co, ho, wo)


def kernel(x, weight, bias, gamma, beta):
    del bias
    return _conv_bn_relu(x, weight, gamma, beta)


# speedup vs baseline: 1.0441x; 1.0441x over previous
import functools

import jax
import jax.numpy as jnp
from jax import lax
from jax.experimental import pallas as pl
from jax.experimental.pallas import tpu as pltpu

_BN_EPS = 1e-5


def _fused_kernel(x_ref, w_ref, g_ref, b_ref, out_ref, yconv_ref, stats_ref,
                  *, n, ho, wo, kh, kw, c, co, m_total):
    p = pl.program_id(0)
    i = pl.program_id(1)
    m = ho * wo

    @pl.when(p == 0)
    def _phase0():
        xt = jnp.transpose(x_ref[0], (1, 2, 0)).astype(jnp.bfloat16)
        xp = jnp.pad(xt, ((1, 1), (1, 1), (0, 0)))
        cat = jnp.concatenate([xp[:, j:j + wo, :] for j in range(kw)], axis=-1)
        acc = None
        for ki in range(kh):
            lhs = cat[ki:ki + ho].reshape(m, kw * c)
            part = jnp.dot(lhs, w_ref[ki], preferred_element_type=jnp.float32)
            acc = part if acc is None else acc + part
        part_stats = jnp.concatenate(
            [jnp.sum(acc, axis=0, keepdims=True),
             jnp.sum(acc * acc, axis=0, keepdims=True)], axis=0)

        @pl.when(i == 0)
        def _():
            stats_ref[...] = part_stats

        @pl.when(i > 0)
        def _():
            stats_ref[...] = stats_ref[...] + part_stats

        yconv_ref[i] = acc.astype(jnp.bfloat16)

    @pl.when(p == 1)
    def _phase1():
        mean = stats_ref[0:1] / m_total
        ex2 = stats_ref[1:2] / m_total
        var = jnp.maximum(ex2 - mean * mean, 0.0)
        scale = g_ref[...] * lax.rsqrt(var + _BN_EPS)
        shift = b_ref[...] - mean * scale
        y = jnp.maximum(yconv_ref[i].astype(jnp.float32) * scale + shift, 0.0)
        out_ref[0] = jnp.transpose(y, (1, 0))


@jax.jit
def _conv_bn_relu(x, weight, gamma, beta):
    n, c, h, w = x.shape
    co, _, kh, kw = weight.shape
    ho, wo = h, w
    m = ho * wo
    m_total = n * m

    w_cat = jnp.transpose(weight, (2, 3, 1, 0)).reshape(kh, kw * c, co)
    w_cat = w_cat.astype(jnp.bfloat16)
    g2 = gamma.reshape(1, co)
    b2 = beta.reshape(1, co)

    out_cm = pl.pallas_call(
        functools.partial(_fused_kernel, n=n, ho=ho, wo=wo, kh=kh, kw=kw,
                          c=c, co=co, m_total=m_total),
        out_shape=jax.ShapeDtypeStruct((n, co, m), jnp.float32),
        grid=(2, n),
        in_specs=[
            pl.BlockSpec((1, c, h, w),
                         lambda p, i: (jnp.where(p == 0, i, n - 1), 0, 0, 0)),
            pl.BlockSpec((kh, kw * c, co), lambda p, i: (0, 0, 0)),
            pl.BlockSpec((1, co), lambda p, i: (0, 0)),
            pl.BlockSpec((1, co), lambda p, i: (0, 0)),
        ],
        out_specs=pl.BlockSpec((1, co, m),
                               lambda p, i: (jnp.where(p == 0, 0, i), 0, 0)),
        scratch_shapes=[pltpu.VMEM((n, m, co), jnp.bfloat16),
                        pltpu.VMEM((2, co), jnp.float32)],
        compiler_params=pltpu.CompilerParams(
            dimension_semantics=("arbitrary", "arbitrary"),
            vmem_limit_bytes=56 * 1024 * 1024,
        ),
    )(x, w_cat, g2, b2)

    return out_cm.reshape(n, co, ho, wo)


def kernel(x, weight, bias, gamma, beta):
    del bias
    return _conv_bn_relu(x, weight, gamma, beta)


# speedup vs baseline: 1.1384x; 1.0903x over previous
import functools

import jax
import jax.numpy as jnp
from jax import lax
from jax.experimental import pallas as pl
from jax.experimental.pallas import tpu as pltpu

_BN_EPS = 1e-5


def _fused_kernel(x_ref, w_ref, g_ref, b_ref, out_ref, yconv_ref, stats_ref,
                  *, n, ho, wo, kh, kw, c, co, m_total):
    p = pl.program_id(0)
    i = pl.program_id(1)
    m = ho * wo

    @pl.when(p == 0)
    def _phase0():
        xt = jnp.transpose(x_ref[0], (1, 0)).astype(jnp.bfloat16)
        xp = jnp.pad(xt.reshape(ho, wo, c), ((1, 1), (1, 1), (0, 0)))
        cat = jnp.concatenate([xp[:, j:j + wo, :] for j in range(kw)], axis=-1)
        acc = None
        for ki in range(kh):
            lhs = cat[ki:ki + ho].reshape(m, kw * c)
            part = jnp.dot(lhs, w_ref[ki], preferred_element_type=jnp.float32)
            acc = part if acc is None else acc + part
        part_stats = jnp.concatenate(
            [jnp.sum(acc, axis=0, keepdims=True),
             jnp.sum(acc * acc, axis=0, keepdims=True)], axis=0)

        @pl.when(i == 0)
        def _():
            stats_ref[...] = part_stats

        @pl.when(i > 0)
        def _():
            stats_ref[...] = stats_ref[...] + part_stats

        yconv_ref[i] = acc.astype(jnp.bfloat16)

    @pl.when(p == 1)
    def _phase1():
        mean = stats_ref[0:1] / m_total
        ex2 = stats_ref[1:2] / m_total
        var = jnp.maximum(ex2 - mean * mean, 0.0)
        scale = g_ref[...] * lax.rsqrt(var + _BN_EPS)
        shift = b_ref[...] - mean * scale
        y = jnp.maximum(yconv_ref[i].astype(jnp.float32) * scale + shift, 0.0)
        out_ref[0] = jnp.transpose(y, (1, 0))


@jax.jit
def _conv_bn_relu(x, weight, gamma, beta):
    n, c, h, w = x.shape
    co, _, kh, kw = weight.shape
    ho, wo = h, w
    m = ho * wo
    m_total = n * m

    w_cat = jnp.transpose(weight, (2, 3, 1, 0)).reshape(kh, kw * c, co)
    w_cat = w_cat.astype(jnp.bfloat16)
    g2 = gamma.reshape(1, co)
    b2 = beta.reshape(1, co)

    out_cm = pl.pallas_call(
        functools.partial(_fused_kernel, n=n, ho=ho, wo=wo, kh=kh, kw=kw,
                          c=c, co=co, m_total=m_total),
        out_shape=jax.ShapeDtypeStruct((n, co, m), jnp.float32),
        grid=(2, n),
        in_specs=[
            pl.BlockSpec((1, c, m),
                         lambda p, i: (jnp.where(p == 0, i, n - 1), 0, 0)),
            pl.BlockSpec((kh, kw * c, co), lambda p, i: (0, 0, 0)),
            pl.BlockSpec((1, co), lambda p, i: (0, 0)),
            pl.BlockSpec((1, co), lambda p, i: (0, 0)),
        ],
        out_specs=pl.BlockSpec((1, co, m),
                               lambda p, i: (jnp.where(p == 0, 0, i), 0, 0)),
        scratch_shapes=[pltpu.VMEM((n, m, co), jnp.bfloat16),
                        pltpu.VMEM((2, co), jnp.float32)],
        compiler_params=pltpu.CompilerParams(
            dimension_semantics=("arbitrary", "arbitrary"),
            vmem_limit_bytes=56 * 1024 * 1024,
        ),
    )(x.reshape(n, c, m), w_cat, g2, b2)

    return out_cm.reshape(n, co, ho, wo)


def kernel(x, weight, bias, gamma, beta):
    del bias
    return _conv_bn_relu(x, weight, gamma, beta)


# speedup vs baseline: 1.3494x; 1.1854x over previous
import functools

import jax
import jax.numpy as jnp
from jax import lax
from jax.experimental import pallas as pl
from jax.experimental.pallas import tpu as pltpu

_BN_EPS = 1e-5


def _fused_kernel(slab_ref, w_ref, g_ref, b_ref, out_ref, yconv_ref,
                  stats_ref, *, n, ho, wo, kh, kw, c, co, m_total):
    p = pl.program_id(0)
    i = pl.program_id(1)
    m = ho * wo

    @pl.when(p == 0)
    def _phase0():
        cat = jnp.concatenate(
            [slab_ref[0, :, j:j + wo, :] for j in range(kw)], axis=-1)
        acc = None
        for ki in range(kh):
            lhs = cat[ki:ki + ho].reshape(m, kw * c)
            part = jnp.dot(lhs, w_ref[ki], preferred_element_type=jnp.float32)
            acc = part if acc is None else acc + part
        part_stats = jnp.concatenate(
            [jnp.sum(acc, axis=0, keepdims=True),
             jnp.sum(acc * acc, axis=0, keepdims=True)], axis=0)

        @pl.when(i == 0)
        def _():
            stats_ref[...] = part_stats

        @pl.when(i > 0)
        def _():
            stats_ref[...] = stats_ref[...] + part_stats

        yconv_ref[i] = acc.astype(jnp.bfloat16)

    @pl.when(p == 1)
    def _phase1():
        mean = stats_ref[0:1] / m_total
        ex2 = stats_ref[1:2] / m_total
        var = jnp.maximum(ex2 - mean * mean, 0.0)
        scale = g_ref[...] * lax.rsqrt(var + _BN_EPS)
        shift = b_ref[...] - mean * scale
        y = jnp.maximum(yconv_ref[i].astype(jnp.float32) * scale + shift, 0.0)
        out_ref[0] = jnp.transpose(y, (1, 0))


@jax.jit
def _conv_bn_relu(x, weight, gamma, beta):
    n, c, h, w = x.shape
    co, _, kh, kw = weight.shape
    ho, wo = h, w
    m = ho * wo
    m_total = n * m

    slab = jnp.pad(jnp.transpose(x, (0, 2, 3, 1)),
                   ((0, 0), (1, 1), (1, 1), (0, 0))).astype(jnp.bfloat16)
    w_cat = jnp.transpose(weight, (2, 3, 1, 0)).reshape(kh, kw * c, co)
    w_cat = w_cat.astype(jnp.bfloat16)
    g2 = gamma.reshape(1, co)
    b2 = beta.reshape(1, co)

    out_cm = pl.pallas_call(
        functools.partial(_fused_kernel, n=n, ho=ho, wo=wo, kh=kh, kw=kw,
                          c=c, co=co, m_total=m_total),
        out_shape=jax.ShapeDtypeStruct((n, co, m), jnp.float32),
        grid=(2, n),
        in_specs=[
            pl.BlockSpec((1, h + kh - 1, w + kw - 1, c),
                         lambda p, i: (jnp.where(p == 0, i, n - 1), 0, 0, 0)),
            pl.BlockSpec((kh, kw * c, co), lambda p, i: (0, 0, 0)),
            pl.BlockSpec((1, co), lambda p, i: (0, 0)),
            pl.BlockSpec((1, co), lambda p, i: (0, 0)),
        ],
        out_specs=pl.BlockSpec((1, co, m),
                               lambda p, i: (jnp.where(p == 0, 0, i), 0, 0)),
        scratch_shapes=[pltpu.VMEM((n, m, co), jnp.bfloat16),
                        pltpu.VMEM((2, co), jnp.float32)],
        compiler_params=pltpu.CompilerParams(
            dimension_semantics=("arbitrary", "arbitrary"),
            vmem_limit_bytes=56 * 1024 * 1024,
        ),
    )(slab, w_cat, g2, b2)

    return out_cm.reshape(n, co, ho, wo)


def kernel(x, weight, bias, gamma, beta):
    del bias
    return _conv_bn_relu(x, weight, gamma, beta)
